# baseline (speedup 1.0000x reference)
import jax
import jax.numpy as jnp
from jax.experimental import pallas as pl
from jax.experimental.pallas import tpu as pltpu

VOCAB = 1_000_000
EMBED = 16
WINDOW = 2
BLK = 32_768


def _body(ctx_ref, emb0_ref, emb1_ref, w_ref, b_ref, out_ref):
    r0 = ctx_ref[0] % 8
    r1 = ctx_ref[1] % 8
    x = jnp.concatenate([emb0_ref[r0, :], emb1_ref[r1, :]])
    acc = jax.lax.dot_general(
        x[None, :], w_ref[...],
        (((1,), (1,)), ((), ())),
        preferred_element_type=jnp.float32,
    )
    out_ref[...] = acc + b_ref[...]


def kernel(context, emb_table, W, b):
    b2 = b.reshape(1, VOCAB)
    grid = (pl.cdiv(VOCAB, BLK),)
    grid_spec = pltpu.PrefetchScalarGridSpec(
        num_scalar_prefetch=1,
        grid=grid,
        in_specs=[
            pl.BlockSpec((8, EMBED), lambda i, ctx: (ctx[0] // 8, 0)),
            pl.BlockSpec((8, EMBED), lambda i, ctx: (ctx[1] // 8, 0)),
            pl.BlockSpec((BLK, EMBED * WINDOW), lambda i, ctx: (i, 0)),
            pl.BlockSpec((1, BLK), lambda i, ctx: (0, i)),
        ],
        out_specs=pl.BlockSpec((1, BLK), lambda i, ctx: (0, i)),
    )
    out = pl.pallas_call(
        _body,
        grid_spec=grid_spec,
        out_shape=jax.ShapeDtypeStruct((1, VOCAB), jnp.float32),
    )(context, emb_table, emb_table, W, b2)
    return out


# speedup vs baseline: 13.0134x; 13.0134x over previous
import jax
import jax.numpy as jnp
from jax.experimental import pallas as pl
from jax.experimental.pallas import tpu as pltpu

VOCAB = 1_000_000
EMBED = 16
WINDOW = 2
BLKC = 32_768


def _body(ctx_ref, embt0_ref, embt1_ref, wt_ref, b_ref, out_ref):
    l0 = ctx_ref[0] % 128
    l1 = ctx_ref[1] % 128
    lane = jax.lax.broadcasted_iota(jnp.int32, (1, 128), 1)
    x0 = jnp.sum(jnp.where(lane == l0, embt0_ref[...], 0.0), axis=1,
                 keepdims=True)
    x1 = jnp.sum(jnp.where(lane == l1, embt1_ref[...], 0.0), axis=1,
                 keepdims=True)
    x = jnp.concatenate([x0, x1], axis=0)
    out_ref[...] = (jnp.sum(wt_ref[...] * x, axis=0, keepdims=True)
                    + b_ref[...][None, :])


def kernel(context, emb_table, W, b):
    wt = W.T
    embt = emb_table.T
    grid = (pl.cdiv(VOCAB, BLKC),)
    grid_spec = pltpu.PrefetchScalarGridSpec(
        num_scalar_prefetch=1,
        grid=grid,
        in_specs=[
            pl.BlockSpec((EMBED, 128), lambda i, ctx: (0, ctx[0] // 128)),
            pl.BlockSpec((EMBED, 128), lambda i, ctx: (0, ctx[1] // 128)),
            pl.BlockSpec((EMBED * WINDOW, BLKC), lambda i, ctx: (0, i)),
            pl.BlockSpec((BLKC,), lambda i, ctx: (i,)),
        ],
        out_specs=pl.BlockSpec((1, BLKC), lambda i, ctx: (0, i)),
    )
    out = pl.pallas_call(
        _body,
        grid_spec=grid_spec,
        out_shape=jax.ShapeDtypeStruct((1, VOCAB), jnp.float32),
    )(context, embt, embt, wt, b)
    return out


# speedup vs baseline: 15.0134x; 1.1537x over previous
import jax
import jax.numpy as jnp
from jax.experimental import pallas as pl
from jax.experimental.pallas import tpu as pltpu

VOCAB = 1_000_000
EMBED = 16
WINDOW = 2
BLKC = 65_536


def _body(ctx_ref, embt0_ref, embt1_ref, wt_ref, b_ref, out_ref):
    l0 = ctx_ref[0] % 128
    l1 = ctx_ref[1] % 128
    lane = jax.lax.broadcasted_iota(jnp.int32, (1, 128), 1)
    x0 = jnp.sum(jnp.where(lane == l0, embt0_ref[...], 0.0), axis=1,
                 keepdims=True)
    x1 = jnp.sum(jnp.where(lane == l1, embt1_ref[...], 0.0), axis=1,
                 keepdims=True)
    x = jnp.concatenate([x0, x1], axis=0)
    out_ref[...] = (jnp.sum(wt_ref[...] * x, axis=0, keepdims=True)
                    + b_ref[...][None, :])


def kernel(context, emb_table, W, b):
    wt = W.T
    embt = emb_table.T
    grid = (pl.cdiv(VOCAB, BLKC),)
    grid_spec = pltpu.PrefetchScalarGridSpec(
        num_scalar_prefetch=1,
        grid=grid,
        in_specs=[
            pl.BlockSpec((EMBED, 128), lambda i, ctx: (0, ctx[0] // 128)),
            pl.BlockSpec((EMBED, 128), lambda i, ctx: (0, ctx[1] // 128)),
            pl.BlockSpec((EMBED * WINDOW, BLKC), lambda i, ctx: (0, i)),
            pl.BlockSpec((BLKC,), lambda i, ctx: (i,)),
        ],
        out_specs=pl.BlockSpec((1, BLKC), lambda i, ctx: (0, i)),
    )
    out = pl.pallas_call(
        _body,
        grid_spec=grid_spec,
        out_shape=jax.ShapeDtypeStruct((1, VOCAB), jnp.float32),
    )(context, embt, embt, wt, b)
    return out


# speedup vs baseline: 15.3594x; 1.0230x over previous
import jax
import jax.numpy as jnp
from jax.experimental import pallas as pl
from jax.experimental.pallas import tpu as pltpu

VOCAB = 1_000_000
EMBED = 16
WINDOW = 2
BLKC = 131_072


def _body(ctx_ref, embt0_ref, embt1_ref, wt_ref, b_ref, out_ref):
    l0 = ctx_ref[0] % 128
    l1 = ctx_ref[1] % 128
    lane = jax.lax.broadcasted_iota(jnp.int32, (1, 128), 1)
    x0 = jnp.sum(jnp.where(lane == l0, embt0_ref[...], 0.0), axis=1,
                 keepdims=True)
    x1 = jnp.sum(jnp.where(lane == l1, embt1_ref[...], 0.0), axis=1,
                 keepdims=True)
    x = jnp.concatenate([x0, x1], axis=0)
    out_ref[...] = (jnp.sum(wt_ref[...] * x, axis=0, keepdims=True)
                    + b_ref[...][None, :])


def kernel(context, emb_table, W, b):
    wt = W.T
    embt = emb_table.T
    grid = (pl.cdiv(VOCAB, BLKC),)
    grid_spec = pltpu.PrefetchScalarGridSpec(
        num_scalar_prefetch=1,
        grid=grid,
        in_specs=[
            pl.BlockSpec((EMBED, 128), lambda i, ctx: (0, ctx[0] // 128)),
            pl.BlockSpec((EMBED, 128), lambda i, ctx: (0, ctx[1] // 128)),
            pl.BlockSpec((EMBED * WINDOW, BLKC), lambda i, ctx: (0, i)),
            pl.BlockSpec((BLKC,), lambda i, ctx: (i,)),
        ],
        out_specs=pl.BlockSpec((1, BLKC), lambda i, ctx: (0, i)),
    )
    out = pl.pallas_call(
        _body,
        grid_spec=grid_spec,
        out_shape=jax.ShapeDtypeStruct((1, VOCAB), jnp.float32),
    )(context, embt, embt, wt, b)
    return out
